# baseline (speedup 1.0000x reference)
import functools

import jax
import jax.numpy as jnp
from jax import lax
from jax.experimental import pallas as pl
from jax.experimental.pallas import tpu as pltpu
from jax.experimental.pallas import tpu_sc as plsc

_NC = 2
_NS = 16
_C = 128
_SBUF = 40
_SPLIT0 = 0.50


def _mm_body(x_ref, w_ref, o_ref):
    o_ref[...] = jnp.dot(x_ref[...], w_ref[...],
                         preferred_element_type=jnp.float32)


def _merge_body(p_ref, o_ref):
    o_ref[...] = p_ref[0] + p_ref[1]


def _bcast_lane(vv, lane):
    return lax.gather(
        vv, jnp.full((16, 1), lane, jnp.int32),
        lax.GatherDimensionNumbers(
            offset_dims=(), collapsed_slice_dims=(0,), start_index_map=(0,)),
        (1,),
        mode=lax.GatherScatterMode.PROMISE_IN_BOUNDS)


@functools.partial(jax.jit, static_argnums=(4, 5))
def _agg(support, rowh, colh, valh, N, D):
    nrows = rowh.shape[0]
    cpt = nrows // (_NC * _NS)
    rpt = (N // _NS) // 8 * 8
    rem = N - _NS * rpt
    mesh = plsc.VectorSubcoreMesh(core_axis_name="c", subcore_axis_name="s")

    csplit0 = int(2 * cpt * _SPLIT0) // 8 * 8

    def _chop(total):
        out = []
        r = total
        while r > 0:
            n = min(_SBUF, r)
            out.append(n)
            r -= n
        return out

    stages0 = _chop(csplit0)
    stages1 = _chop(2 * cpt - csplit0)

    def scale(valb, gbuf, k, b):
        @plsc.parallel_loop(0, _C // 16, step=1)
        def group(g):
            vv = valb[k, pl.ds(g * 16, 16)]
            for lane in range(16):
                vb = _bcast_lane(vv, lane)
                for j in range(D // 16):
                    sl = pl.ds(j * 16, 16)
                    e = g * 16 + lane
                    gbuf[b, e, sl] = gbuf[b, e, sl] * vb

    def body(sup, rowh_r, colh_r, valh_r, out,
             colb, rowb, valb, gbuf, acc, gsem0, gsem1, ssem):
        gsems = (gsem0, gsem1)
        cid = lax.axis_index("c")
        sid = lax.axis_index("s")

        def zrow(r, carry):
            for j in range(D // 16):
                gbuf[0, r, pl.ds(j * 16, 16)] = jnp.zeros((16,), jnp.float32)
            return carry

        lax.fori_loop(0, _C, zrow, 0)
        for i in range(rpt // _C):
            pltpu.sync_copy(gbuf.at[0],
                            acc.at[pl.ds(sid * rpt + i * _C, _C)])
        tail = rpt % _C
        if tail:
            pltpu.sync_copy(gbuf.at[0, pl.ds(0, tail)],
                            acc.at[pl.ds(sid * rpt + rpt - tail, tail)])
        if rem:
            @pl.when(sid == 0)
            def _():
                pltpu.sync_copy(gbuf.at[0, pl.ds(0, rem)],
                                acc.at[pl.ds(_NS * rpt, rem)])
        plsc.subcore_barrier()

        def step(k, b, n):
            @pl.when(k >= 1)
            def _():
                km = jnp.maximum(k - 1, 0)
                pltpu.make_async_copy(
                    gbuf.at[1 - b], acc.at[rowb.at[km]], ssem).wait()

            @pl.when(k + 1 < n)
            def _():
                kn = jnp.minimum(k + 1, n - 1)
                pltpu.async_copy(sup.at[colb.at[kn]], gbuf.at[1 - b],
                                 gsems[1 - b])

            pltpu.make_async_copy(sup.at[colb.at[k]], gbuf.at[b],
                                  gsems[b]).wait()
            scale(valb, gbuf, k, b)
            pltpu.async_copy(gbuf.at[b], acc.at[rowb.at[k]], ssem, add=True)

        def stage(base, n):
            pltpu.sync_copy(colh_r.at[pl.ds(base, n)], colb.at[pl.ds(0, n)])
            pltpu.sync_copy(rowh_r.at[pl.ds(base, n)], rowb.at[pl.ds(0, n)])
            pltpu.sync_copy(valh_r.at[pl.ds(base, n)], valb.at[pl.ds(0, n)])
            pltpu.async_copy(sup.at[colb.at[0]], gbuf.at[0], gsems[0])

            def pair(k2, carry):
                step(k2 * 2, 0, n)
                step(k2 * 2 + 1, 1, n)
                return carry

            lax.fori_loop(0, n // 2, pair, 0)
            pltpu.make_async_copy(
                gbuf.at[1], acc.at[rowb.at[n - 1]], ssem).wait()

        pairbase = sid * (2 * cpt)

        @pl.when(cid == 0)
        def _():
            off = 0
            for n in stages0:
                stage(pairbase + off, n)
                off += n

        @pl.when(cid == 1)
        def _():
            off = csplit0
            for n in stages1:
                stage(pairbase + off, n)
                off += n

        plsc.subcore_barrier()
        pltpu.sync_copy(acc.at[pl.ds(sid * rpt, rpt)],
                        out.at[cid, pl.ds(sid * rpt, rpt)])
        if rem:
            @pl.when(sid == 0)
            def _():
                pltpu.sync_copy(acc.at[pl.ds(_NS * rpt, rem)],
                                out.at[cid, pl.ds(_NS * rpt, rem)])

    agg = pl.kernel(
        body,
        out_type=jax.ShapeDtypeStruct((_NC, N, D), jnp.float32),
        mesh=mesh,
        scratch_types=[
            pltpu.VMEM((_SBUF, _C), jnp.int32),
            pltpu.VMEM((_SBUF, _C), jnp.int32),
            pltpu.VMEM((_SBUF, _C), jnp.float32),
            pltpu.VMEM((2, _C, D), jnp.float32),
            pltpu.VMEM_SHARED((N, D), jnp.float32),
            pltpu.SemaphoreType.DMA,
            pltpu.SemaphoreType.DMA,
            pltpu.SemaphoreType.DMA,
        ],
    )
    return agg(support, rowh, colh, valh)


def kernel(x, edge_index, adj_values, W):
    N, _ = x.shape
    D = W.shape[1]
    E = adj_values.shape[0]
    rb = N // 5

    support = pl.pallas_call(
        _mm_body,
        grid=(5,),
        in_specs=[
            pl.BlockSpec((rb, x.shape[1]), lambda i: (i, 0)),
            pl.BlockSpec(W.shape, lambda i: (0, 0)),
        ],
        out_specs=pl.BlockSpec((rb, D), lambda i: (i, 0)),
        out_shape=jax.ShapeDtypeStruct((N, D), jnp.float32),
    )(x, W)

    nt = _NC * _NS
    cpt = -(-E // (_C * nt))
    cpt = (cpt + 15) // 16 * 16
    pad = _C * nt * cpt - E
    spread = (jnp.arange(pad, dtype=jnp.int32) * 97) % N
    rowp = jnp.concatenate([edge_index[0], spread]).reshape(nt * cpt, _C)
    colp = jnp.concatenate([edge_index[1], spread]).reshape(nt * cpt, _C)
    valp = jnp.pad(adj_values, (0, pad)).reshape(nt * cpt, _C)

    partial = _agg(support, rowp, colp, valp, N, D)

    out = pl.pallas_call(
        _merge_body,
        grid=(5,),
        in_specs=[pl.BlockSpec((_NC, rb, D), lambda i: (0, i, 0))],
        out_specs=pl.BlockSpec((rb, D), lambda i: (i, 0)),
        out_shape=jax.ShapeDtypeStruct((N, D), jnp.float32),
    )(partial)
    return out


# speedup vs baseline: 1.0142x; 1.0142x over previous
import functools

import jax
import jax.numpy as jnp
from jax import lax
from jax.experimental import pallas as pl
from jax.experimental.pallas import tpu as pltpu
from jax.experimental.pallas import tpu_sc as plsc

_NC = 2
_NS = 16
_C = 128
_SBUF = 40
_SPLIT0 = 0.50


def _mm_body(x_ref, w_ref, o_ref):
    o_ref[...] = jnp.dot(x_ref[...], w_ref[...],
                         preferred_element_type=jnp.float32)


def _merge_body(p_ref, o_ref):
    o_ref[...] = p_ref[0] + p_ref[1]


def _bcast_lane(vv, lane):
    return lax.gather(
        vv, jnp.full((16, 1), lane, jnp.int32),
        lax.GatherDimensionNumbers(
            offset_dims=(), collapsed_slice_dims=(0,), start_index_map=(0,)),
        (1,),
        mode=lax.GatherScatterMode.PROMISE_IN_BOUNDS)


@functools.partial(jax.jit, static_argnums=(4, 5))
def _agg(support, rowh, colh, valh, N, D):
    nrows = rowh.shape[0]
    cpt = nrows // (_NC * _NS)
    rpt = (N // _NS) // 8 * 8
    rem = N - _NS * rpt
    mesh = plsc.VectorSubcoreMesh(core_axis_name="c", subcore_axis_name="s")

    csplit0 = int(2 * cpt * _SPLIT0) // 8 * 8

    def _chop(total):
        out = []
        r = total
        while r > 0:
            n = min(_SBUF, r)
            out.append(n)
            r -= n
        return out

    stages0 = _chop(csplit0)
    stages1 = _chop(2 * cpt - csplit0)

    def scale(valb, gbuf, k, b):
        def group(g, carry):
            vv = valb[k, pl.ds(g * 16, 16)]
            for lane in range(16):
                vb = _bcast_lane(vv, lane)
                for j in range(D // 16):
                    sl = pl.ds(j * 16, 16)
                    e = g * 16 + lane
                    gbuf[b, e, sl] = gbuf[b, e, sl] * vb
            return carry

        lax.fori_loop(0, _C // 16, group, 0)

    def body(sup, rowh_r, colh_r, valh_r, out,
             colb, rowb, valb, gbuf, acc, gsem0, gsem1, ssem):
        gsems = (gsem0, gsem1)
        cid = lax.axis_index("c")
        sid = lax.axis_index("s")

        def zrow(r, carry):
            for j in range(D // 16):
                gbuf[0, r, pl.ds(j * 16, 16)] = jnp.zeros((16,), jnp.float32)
            return carry

        lax.fori_loop(0, _C, zrow, 0)
        for i in range(rpt // _C):
            pltpu.sync_copy(gbuf.at[0],
                            acc.at[pl.ds(sid * rpt + i * _C, _C)])
        tail = rpt % _C
        if tail:
            pltpu.sync_copy(gbuf.at[0, pl.ds(0, tail)],
                            acc.at[pl.ds(sid * rpt + rpt - tail, tail)])
        if rem:
            @pl.when(sid == 0)
            def _():
                pltpu.sync_copy(gbuf.at[0, pl.ds(0, rem)],
                                acc.at[pl.ds(_NS * rpt, rem)])
        plsc.subcore_barrier()

        def step(k, b, n):
            @pl.when(k >= 1)
            def _():
                km = jnp.maximum(k - 1, 0)
                pltpu.make_async_copy(
                    gbuf.at[1 - b], acc.at[rowb.at[km]], ssem).wait()

            @pl.when(k + 1 < n)
            def _():
                kn = jnp.minimum(k + 1, n - 1)
                pltpu.async_copy(sup.at[colb.at[kn]], gbuf.at[1 - b],
                                 gsems[1 - b])

            pltpu.make_async_copy(sup.at[colb.at[k]], gbuf.at[b],
                                  gsems[b]).wait()
            scale(valb, gbuf, k, b)
            pltpu.async_copy(gbuf.at[b], acc.at[rowb.at[k]], ssem, add=True)

        def stage(base, n):
            pltpu.sync_copy(colh_r.at[pl.ds(base, n)], colb.at[pl.ds(0, n)])
            pltpu.sync_copy(rowh_r.at[pl.ds(base, n)], rowb.at[pl.ds(0, n)])
            pltpu.sync_copy(valh_r.at[pl.ds(base, n)], valb.at[pl.ds(0, n)])
            pltpu.async_copy(sup.at[colb.at[0]], gbuf.at[0], gsems[0])

            def pair(k2, carry):
                step(k2 * 2, 0, n)
                step(k2 * 2 + 1, 1, n)
                return carry

            lax.fori_loop(0, n // 2, pair, 0)
            pltpu.make_async_copy(
                gbuf.at[1], acc.at[rowb.at[n - 1]], ssem).wait()

        pairbase = sid * (2 * cpt)

        @pl.when(cid == 0)
        def _():
            off = 0
            for n in stages0:
                stage(pairbase + off, n)
                off += n

        @pl.when(cid == 1)
        def _():
            off = csplit0
            for n in stages1:
                stage(pairbase + off, n)
                off += n

        plsc.subcore_barrier()
        pltpu.sync_copy(acc.at[pl.ds(sid * rpt, rpt)],
                        out.at[cid, pl.ds(sid * rpt, rpt)])
        if rem:
            @pl.when(sid == 0)
            def _():
                pltpu.sync_copy(acc.at[pl.ds(_NS * rpt, rem)],
                                out.at[cid, pl.ds(_NS * rpt, rem)])

    agg = pl.kernel(
        body,
        out_type=jax.ShapeDtypeStruct((_NC, N, D), jnp.float32),
        mesh=mesh,
        scratch_types=[
            pltpu.VMEM((_SBUF, _C), jnp.int32),
            pltpu.VMEM((_SBUF, _C), jnp.int32),
            pltpu.VMEM((_SBUF, _C), jnp.float32),
            pltpu.VMEM((2, _C, D), jnp.float32),
            pltpu.VMEM_SHARED((N, D), jnp.float32),
            pltpu.SemaphoreType.DMA,
            pltpu.SemaphoreType.DMA,
            pltpu.SemaphoreType.DMA,
        ],
    )
    return agg(support, rowh, colh, valh)


def kernel(x, edge_index, adj_values, W):
    N, _ = x.shape
    D = W.shape[1]
    E = adj_values.shape[0]
    rb = N // 5

    support = pl.pallas_call(
        _mm_body,
        grid=(5,),
        in_specs=[
            pl.BlockSpec((rb, x.shape[1]), lambda i: (i, 0)),
            pl.BlockSpec(W.shape, lambda i: (0, 0)),
        ],
        out_specs=pl.BlockSpec((rb, D), lambda i: (i, 0)),
        out_shape=jax.ShapeDtypeStruct((N, D), jnp.float32),
    )(x, W)

    nt = _NC * _NS
    cpt = -(-E // (_C * nt))
    cpt = (cpt + 15) // 16 * 16
    pad = _C * nt * cpt - E
    spread = (jnp.arange(pad, dtype=jnp.int32) * 97) % N
    rowp = jnp.concatenate([edge_index[0], spread]).reshape(nt * cpt, _C)
    colp = jnp.concatenate([edge_index[1], spread]).reshape(nt * cpt, _C)
    valp = jnp.pad(adj_values, (0, pad)).reshape(nt * cpt, _C)

    partial = _agg(support, rowp, colp, valp, N, D)

    out = pl.pallas_call(
        _merge_body,
        grid=(5,),
        in_specs=[pl.BlockSpec((_NC, rb, D), lambda i: (0, i, 0))],
        out_specs=pl.BlockSpec((rb, D), lambda i: (i, 0)),
        out_shape=jax.ShapeDtypeStruct((N, D), jnp.float32),
    )(partial)
    return out
